# baseline (speedup 1.0000x reference)
import functools

import jax
import jax.numpy as jnp
from jax import lax
from jax.experimental import pallas as pl
from jax.experimental.pallas import tpu as pltpu
from jax.experimental.pallas import tpu_sc as plsc

N = 10000
E = 50000
D_IN = 32
D_EDGE = 32
H = 64
F1 = 128

NC = 2
NS = 16
NW = NC * NS

SUB = 112
NSUB = 14
CHUNK = NSUB * SUB
E_PAD = NW * CHUNK
BE = 512

N_PAD = 10240
TILE_N = N_PAD // NS
BN = 1000


def _edge_body(ea_ref, xs_ref, w1t_ref, b1_ref, wbigt_ref, bmt_ref, out_ref, *, d_in):
    ea = ea_ref[...]
    xs = xs_ref[...].T
    h = jnp.dot(w1t_ref[...], ea, preferred_element_type=jnp.float32)
    h = jnp.maximum(h + b1_ref[...], 0.0)
    y = (h[:, None, :] * xs[None, :, :]).reshape(64 * d_in, BE)
    m = jnp.dot(wbigt_ref[...], y, preferred_element_type=jnp.float32)
    m = m + jnp.dot(bmt_ref[...], xs, preferred_element_type=jnp.float32)
    out_ref[...] = m.T


def _edge_call(ea, xs, w1t, b1c, wbigt, bmt, d_in):
    grid = (E_PAD // BE,)
    return pl.pallas_call(
        functools.partial(_edge_body, d_in=d_in),
        grid=grid,
        in_specs=[
            pl.BlockSpec((D_EDGE, BE), lambda i: (0, i)),
            pl.BlockSpec((BE, d_in), lambda i: (i, 0)),
            pl.BlockSpec((64, D_EDGE), lambda i: (0, 0)),
            pl.BlockSpec((64, 1), lambda i: (0, 0)),
            pl.BlockSpec((64, 64 * d_in), lambda i: (0, 0)),
            pl.BlockSpec((64, d_in), lambda i: (0, 0)),
        ],
        out_specs=pl.BlockSpec((BE, 64), lambda i: (i, 0)),
        out_shape=jax.ShapeDtypeStruct((E_PAD, 64), jnp.float32),
    )(ea, xs, w1t, b1c, wbigt, bmt)


def _elu(v):
    return jnp.where(v > 0.0, v, jnp.exp(jnp.minimum(v, 0.0)) - 1.0)


def _node1_body(s0, c0, x, root, b, out):
    cnt = jnp.max(c0[...], axis=1, keepdims=True)
    mean = s0[...] / jnp.maximum(cnt, 1.0)
    v = mean + jnp.dot(x[...], root[...], preferred_element_type=jnp.float32) + b[...]
    out[...] = _elu(v)


def _node1_call(s0, c0, x, root1, b1r):
    grid = (N // BN,)
    f = pl.BlockSpec((BN, 64), lambda i: (i, 0))
    g = pl.BlockSpec((BN, 16), lambda i: (i, 0))
    return pl.pallas_call(
        _node1_body,
        grid=grid,
        in_specs=[f, g,
                  pl.BlockSpec((BN, D_IN), lambda i: (i, 0)),
                  pl.BlockSpec((D_IN, 64), lambda i: (0, 0)),
                  pl.BlockSpec((1, 64), lambda i: (0, 0))],
        out_specs=f,
        out_shape=jax.ShapeDtypeStruct((N, 64), jnp.float32),
    )(s0, c0, x, root1, b1r)


def _node2_body(s0, c0, h, root, b, wf, bf, out):
    cnt = jnp.max(c0[...], axis=1, keepdims=True)
    mean = s0[...] / jnp.maximum(cnt, 1.0)
    v = mean + jnp.dot(h[...], root[...], preferred_element_type=jnp.float32) + b[...]
    t = _elu(v)
    u = jnp.dot(t, wf[...], preferred_element_type=jnp.float32) + bf[...]
    out[...] = _elu(u)


def _node2_call(s0, c0, h1, root2, b2r, wf, bfr):
    grid = (N // BN,)
    f = pl.BlockSpec((BN, 64), lambda i: (i, 0))
    g = pl.BlockSpec((BN, 16), lambda i: (i, 0))
    return pl.pallas_call(
        _node2_body,
        grid=grid,
        in_specs=[f, g, f,
                  pl.BlockSpec((64, 64), lambda i: (0, 0)),
                  pl.BlockSpec((1, 64), lambda i: (0, 0)),
                  pl.BlockSpec((64, F1), lambda i: (0, 0)),
                  pl.BlockSpec((1, F1), lambda i: (0, 0))],
        out_specs=pl.BlockSpec((BN, F1), lambda i: (i, 0)),
        out_shape=jax.ShapeDtypeStruct((N, F1), jnp.float32),
    )(s0, c0, h1, root2, b2r, wf, bfr)


@functools.lru_cache(maxsize=None)
def _make_gather(d):
    mesh = plsc.VectorSubcoreMesh(core_axis_name="c", subcore_axis_name="s",
                                  num_cores=NC, num_subcores=NS)

    @functools.partial(
        pl.kernel,
        out_type=pltpu.HBM((E_PAD, d), jnp.float32),
        mesh=mesh,
        compiler_params=pltpu.CompilerParams(use_tc_tiling_on_sc=False),
        scratch_types=[
            pltpu.VMEM((CHUNK,), jnp.int32),
            pltpu.VMEM((CHUNK, d), jnp.float32),
            pltpu.SemaphoreType.DMA,
            pltpu.SemaphoreType.DMA,
        ],
    )
    def gather_k(tab_hbm, idx_hbm, out_hbm, idx_v, rows_v, gsem, wsem):
        c = lax.axis_index("c")
        s = lax.axis_index("s")
        wid = s * NC + c
        base = wid * CHUNK
        pltpu.sync_copy(idx_hbm.at[pl.ds(base, CHUNK)], idx_v)
        npipe = 4
        sub = CHUNK // npipe
        gets = [
            pltpu.async_copy(tab_hbm.at[idx_v.at[pl.ds(j * sub, sub)]],
                             rows_v.at[pl.ds(j * sub, sub)], gsem)
            for j in range(npipe)
        ]
        puts = []
        for j in range(npipe):
            gets[j].wait()
            puts.append(pltpu.async_copy(
                rows_v.at[pl.ds(j * sub, sub)],
                out_hbm.at[pl.ds(base + j * sub, sub)], wsem))
        for p in puts:
            p.wait()

    return gather_k


NSUB_SC = E_PAD // NS // SUB
QTR = NSUB_SC // 4


@functools.lru_cache(maxsize=None)
def _make_scatter(with_cnt):
    mesh = plsc.VectorSubcoreMesh(core_axis_name="c", subcore_axis_name="s",
                                  num_cores=1, num_subcores=NS)
    out_type = [pltpu.HBM((N_PAD, 64), jnp.float32)]
    scratch = [
        pltpu.VMEM((NSUB_SC, SUB), jnp.int32),
        pltpu.VMEM((QTR, SUB, 64), jnp.float32),
        pltpu.VMEM_SHARED((N_PAD, 64), jnp.float32),
    ]
    if with_cnt:
        out_type.append(pltpu.HBM((N_PAD, 16), jnp.float32))
        scratch.append(pltpu.VMEM((SUB, 16), jnp.float32))
        scratch.append(pltpu.VMEM_SHARED((N_PAD, 16), jnp.float32))

    @functools.partial(pl.kernel, out_type=out_type, mesh=mesh,
                       compiler_params=pltpu.CompilerParams(
                           use_tc_tiling_on_sc=False),
                       scratch_types=scratch)
    def scatter_k(msg_hbm, idx_hbm, zeros_hbm, zeros16_hbm, ones_hbm, *refs):
        if with_cnt:
            out_hbm, cnt_hbm, idx_v, msg_v, acc, ones_v, cacc = refs
        else:
            out_hbm, idx_v, msg_v, acc = refs
        s = lax.axis_index("s")
        pltpu.sync_copy(zeros_hbm, acc.at[pl.ds(s * TILE_N, TILE_N)])
        if with_cnt:
            pltpu.sync_copy(zeros16_hbm, cacc.at[pl.ds(s * TILE_N, TILE_N)])
            pltpu.sync_copy(ones_hbm, ones_v)
        pltpu.sync_copy(idx_hbm.at[s], idx_v)
        plsc.subcore_barrier()
        for q in range(4):
            pltpu.sync_copy(msg_hbm.at[pl.ds(s * NSUB_SC + q * QTR, QTR)],
                            msg_v)

            @pl.loop(0, QTR)
            def _scatter_one(j):
                pltpu.sync_copy(msg_v.at[j],
                                acc.at[idx_v.at[q * QTR + j]], add=True)
                if with_cnt:
                    pltpu.sync_copy(ones_v,
                                    cacc.at[idx_v.at[q * QTR + j]],
                                    add=True)
        plsc.subcore_barrier()
        pltpu.sync_copy(acc.at[pl.ds(s * TILE_N, TILE_N)],
                        out_hbm.at[pl.ds(s * TILE_N, TILE_N)])
        if with_cnt:
            pltpu.sync_copy(cacc.at[pl.ds(s * TILE_N, TILE_N)],
                            cnt_hbm.at[pl.ds(s * TILE_N, TILE_N)])

    return scatter_k


def kernel(x, edge_index, edge_attr, W1a, b1a, W2a, b2a, root1, bias1,
           W1b, b1b, W2b, b2b, root2, bias2, Wf, bf):
    src = edge_index[0].astype(jnp.int32)
    dst = edge_index[1].astype(jnp.int32)
    src2 = jnp.concatenate([src, jnp.zeros((E_PAD - E,), jnp.int32)])
    dst_sc = jnp.concatenate(
        [dst, jnp.full((E_PAD - E,), N, jnp.int32)]).reshape(NS, NSUB_SC, SUB)
    ea_t = jnp.pad(edge_attr, ((0, E_PAD - E), (0, 0))).T
    zeros64 = jnp.zeros((TILE_N, 64), jnp.float32)
    zeros16 = jnp.zeros((TILE_N, 16), jnp.float32)
    ones16 = jnp.ones((SUB, 16), jnp.float32)

    w1t_a = W1a.T
    b1c_a = b1a.reshape(64, 1)
    wbigt_a = W2a.reshape(64 * D_IN, H).T
    bmt_a = b2a.reshape(D_IN, H).T
    w1t_b = W1b.T
    b1c_b = b1b.reshape(64, 1)
    wbigt_b = W2b.reshape(64 * H, H).T
    bmt_b = b2b.reshape(H, H).T

    xs = _make_gather(D_IN)(x, src2)
    msg = _edge_call(ea_t, xs, w1t_a, b1c_a, wbigt_a, bmt_a, D_IN)
    msg3 = msg.reshape(NS * NSUB_SC, SUB, 64)
    sums1, cnts = _make_scatter(True)(msg3, dst_sc, zeros64, zeros16, ones16)
    h1 = _node1_call(sums1, cnts, x, root1, bias1.reshape(1, 64))

    hs = _make_gather(H)(h1, src2)
    msg2 = _edge_call(ea_t, hs, w1t_b, b1c_b, wbigt_b, bmt_b, H)
    msg3b = msg2.reshape(NS * NSUB_SC, SUB, 64)
    sums2, _ = _make_scatter(True)(msg3b, dst_sc, zeros64, zeros16, ones16)
    out = _node2_call(sums2, cnts, h1, root2, bias2.reshape(1, 64),
                      Wf, bf.reshape(1, F1))
    return out


# speedup vs baseline: 1.1474x; 1.1474x over previous
import functools

import jax
import jax.numpy as jnp
from jax import lax
from jax.experimental import pallas as pl
from jax.experimental.pallas import tpu as pltpu
from jax.experimental.pallas import tpu_sc as plsc

N = 10000
E = 50000
D_IN = 32
D_EDGE = 32
H = 64
F1 = 128

NC = 2
NS = 16
NW = NC * NS

SUB = 112
NSUB = 14
CHUNK = NSUB * SUB
E_PAD = NW * CHUNK
BE = 1024

N_PAD = 10240
TILE_N = N_PAD // NS
BN = 1000


def _edge_body(ea_ref, xs_ref, w1t_ref, b1_ref, wbigt_ref, bmt_ref, out_ref, *, d_in):
    ea = ea_ref[...]
    xs = xs_ref[...].T
    h = jnp.dot(w1t_ref[...], ea, preferred_element_type=jnp.float32)
    h = jnp.maximum(h + b1_ref[...], 0.0)
    y = (h[:, None, :] * xs[None, :, :]).reshape(64 * d_in, BE)
    m = jnp.dot(wbigt_ref[...], y, preferred_element_type=jnp.float32)
    m = m + jnp.dot(bmt_ref[...], xs, preferred_element_type=jnp.float32)
    out_ref[...] = m.T


def _edge_call(ea, xs, w1t, b1c, wbigt, bmt, d_in):
    grid = (E_PAD // BE,)
    return pl.pallas_call(
        functools.partial(_edge_body, d_in=d_in),
        grid=grid,
        in_specs=[
            pl.BlockSpec((D_EDGE, BE), lambda i: (0, i)),
            pl.BlockSpec((BE, d_in), lambda i: (i, 0)),
            pl.BlockSpec((64, D_EDGE), lambda i: (0, 0)),
            pl.BlockSpec((64, 1), lambda i: (0, 0)),
            pl.BlockSpec((64, 64 * d_in), lambda i: (0, 0)),
            pl.BlockSpec((64, d_in), lambda i: (0, 0)),
        ],
        out_specs=pl.BlockSpec((BE, 64), lambda i: (i, 0)),
        out_shape=jax.ShapeDtypeStruct((E_PAD, 64), jnp.float32),
    )(ea, xs, w1t, b1c, wbigt, bmt)


def _elu(v):
    return jnp.where(v > 0.0, v, jnp.exp(jnp.minimum(v, 0.0)) - 1.0)


def _node1_body(s0, c0, x, root, b, out):
    cnt = jnp.max(c0[...], axis=1, keepdims=True)
    mean = s0[...] / jnp.maximum(cnt, 1.0)
    v = mean + jnp.dot(x[...], root[...], preferred_element_type=jnp.float32) + b[...]
    out[...] = _elu(v)


def _node1_call(s0, c0, x, root1, b1r):
    grid = (N // BN,)
    f = pl.BlockSpec((BN, 64), lambda i: (i, 0))
    g = pl.BlockSpec((BN, 16), lambda i: (i, 0))
    return pl.pallas_call(
        _node1_body,
        grid=grid,
        in_specs=[f, g,
                  pl.BlockSpec((BN, D_IN), lambda i: (i, 0)),
                  pl.BlockSpec((D_IN, 64), lambda i: (0, 0)),
                  pl.BlockSpec((1, 64), lambda i: (0, 0))],
        out_specs=f,
        out_shape=jax.ShapeDtypeStruct((N, 64), jnp.float32),
    )(s0, c0, x, root1, b1r)


def _node2_body(s0, c0, h, root, b, wf, bf, out):
    cnt = jnp.max(c0[...], axis=1, keepdims=True)
    mean = s0[...] / jnp.maximum(cnt, 1.0)
    v = mean + jnp.dot(h[...], root[...], preferred_element_type=jnp.float32) + b[...]
    t = _elu(v)
    u = jnp.dot(t, wf[...], preferred_element_type=jnp.float32) + bf[...]
    out[...] = _elu(u)


def _node2_call(s0, c0, h1, root2, b2r, wf, bfr):
    grid = (N // BN,)
    f = pl.BlockSpec((BN, 64), lambda i: (i, 0))
    g = pl.BlockSpec((BN, 16), lambda i: (i, 0))
    return pl.pallas_call(
        _node2_body,
        grid=grid,
        in_specs=[f, g, f,
                  pl.BlockSpec((64, 64), lambda i: (0, 0)),
                  pl.BlockSpec((1, 64), lambda i: (0, 0)),
                  pl.BlockSpec((64, F1), lambda i: (0, 0)),
                  pl.BlockSpec((1, F1), lambda i: (0, 0))],
        out_specs=pl.BlockSpec((BN, F1), lambda i: (i, 0)),
        out_shape=jax.ShapeDtypeStruct((N, F1), jnp.float32),
    )(s0, c0, h1, root2, b2r, wf, bfr)


@functools.lru_cache(maxsize=None)
def _make_gather(d):
    mesh = plsc.VectorSubcoreMesh(core_axis_name="c", subcore_axis_name="s",
                                  num_cores=NC, num_subcores=NS)

    @functools.partial(
        pl.kernel,
        out_type=pltpu.HBM((E_PAD, d), jnp.float32),
        mesh=mesh,
        compiler_params=pltpu.CompilerParams(use_tc_tiling_on_sc=False),
        scratch_types=[
            pltpu.VMEM((CHUNK,), jnp.int32),
            pltpu.VMEM((CHUNK, d), jnp.float32),
            pltpu.SemaphoreType.DMA,
            pltpu.SemaphoreType.DMA,
        ],
    )
    def gather_k(tab_hbm, idx_hbm, out_hbm, idx_v, rows_v, gsem, wsem):
        c = lax.axis_index("c")
        s = lax.axis_index("s")
        wid = s * NC + c
        base = wid * CHUNK
        pltpu.sync_copy(idx_hbm.at[pl.ds(base, CHUNK)], idx_v)
        npipe = 4
        sub = CHUNK // npipe
        gets = [
            pltpu.async_copy(tab_hbm.at[idx_v.at[pl.ds(j * sub, sub)]],
                             rows_v.at[pl.ds(j * sub, sub)], gsem)
            for j in range(npipe)
        ]
        puts = []
        for j in range(npipe):
            gets[j].wait()
            puts.append(pltpu.async_copy(
                rows_v.at[pl.ds(j * sub, sub)],
                out_hbm.at[pl.ds(base + j * sub, sub)], wsem))
        for p in puts:
            p.wait()

    return gather_k


NSUB_SC = E_PAD // NS // SUB
QTR = NSUB_SC // 4


@functools.lru_cache(maxsize=None)
def _make_scatter(with_cnt):
    mesh = plsc.VectorSubcoreMesh(core_axis_name="c", subcore_axis_name="s",
                                  num_cores=1, num_subcores=NS)
    out_type = [pltpu.HBM((N_PAD, 64), jnp.float32)]
    scratch = [
        pltpu.VMEM((NSUB_SC, SUB), jnp.int32),
        pltpu.VMEM((QTR, SUB, 64), jnp.float32),
        pltpu.VMEM_SHARED((N_PAD, 64), jnp.float32),
    ]
    if with_cnt:
        out_type.append(pltpu.HBM((N_PAD, 16), jnp.float32))
        scratch.append(pltpu.VMEM((SUB, 16), jnp.float32))
        scratch.append(pltpu.VMEM_SHARED((N_PAD, 16), jnp.float32))

    @functools.partial(pl.kernel, out_type=out_type, mesh=mesh,
                       compiler_params=pltpu.CompilerParams(
                           use_tc_tiling_on_sc=False),
                       scratch_types=scratch)
    def scatter_k(msg_hbm, idx_hbm, zeros_hbm, zeros16_hbm, ones_hbm, *refs):
        if with_cnt:
            out_hbm, cnt_hbm, idx_v, msg_v, acc, ones_v, cacc = refs
        else:
            out_hbm, idx_v, msg_v, acc = refs
        s = lax.axis_index("s")
        pltpu.sync_copy(zeros_hbm, acc.at[pl.ds(s * TILE_N, TILE_N)])
        if with_cnt:
            pltpu.sync_copy(zeros16_hbm, cacc.at[pl.ds(s * TILE_N, TILE_N)])
            pltpu.sync_copy(ones_hbm, ones_v)
        pltpu.sync_copy(idx_hbm.at[s], idx_v)
        plsc.subcore_barrier()
        for q in range(4):
            pltpu.sync_copy(msg_hbm.at[pl.ds(s * NSUB_SC + q * QTR, QTR)],
                            msg_v)

            @pl.loop(0, QTR)
            def _scatter_one(j):
                pltpu.sync_copy(msg_v.at[j],
                                acc.at[idx_v.at[q * QTR + j]], add=True)
                if with_cnt:
                    pltpu.sync_copy(ones_v,
                                    cacc.at[idx_v.at[q * QTR + j]],
                                    add=True)
        plsc.subcore_barrier()
        pltpu.sync_copy(acc.at[pl.ds(s * TILE_N, TILE_N)],
                        out_hbm.at[pl.ds(s * TILE_N, TILE_N)])
        if with_cnt:
            pltpu.sync_copy(cacc.at[pl.ds(s * TILE_N, TILE_N)],
                            cnt_hbm.at[pl.ds(s * TILE_N, TILE_N)])

    return scatter_k


def kernel(x, edge_index, edge_attr, W1a, b1a, W2a, b2a, root1, bias1,
           W1b, b1b, W2b, b2b, root2, bias2, Wf, bf):
    src = edge_index[0].astype(jnp.int32)
    dst = edge_index[1].astype(jnp.int32)
    src2 = jnp.concatenate([src, jnp.zeros((E_PAD - E,), jnp.int32)])
    dst_sc = jnp.concatenate(
        [dst, jnp.full((E_PAD - E,), N, jnp.int32)]).reshape(NS, NSUB_SC, SUB)
    ea_t = jnp.pad(edge_attr, ((0, E_PAD - E), (0, 0))).T
    zeros64 = jnp.zeros((TILE_N, 64), jnp.float32)
    zeros16 = jnp.zeros((TILE_N, 16), jnp.float32)
    ones16 = jnp.ones((SUB, 16), jnp.float32)

    w1t_a = W1a.T
    b1c_a = b1a.reshape(64, 1)
    wbigt_a = W2a.reshape(64 * D_IN, H).T
    bmt_a = b2a.reshape(D_IN, H).T
    w1t_b = W1b.T
    b1c_b = b1b.reshape(64, 1)
    wbigt_b = W2b.reshape(64 * H, H).T
    bmt_b = b2b.reshape(H, H).T

    xs = _make_gather(D_IN)(x, src2)
    msg = _edge_call(ea_t, xs, w1t_a, b1c_a, wbigt_a, bmt_a, D_IN)
    msg3 = msg.reshape(NS * NSUB_SC, SUB, 64)
    sums1, cnts = _make_scatter(True)(msg3, dst_sc, zeros64, zeros16, ones16)
    h1 = _node1_call(sums1, cnts, x, root1, bias1.reshape(1, 64))

    hs = _make_gather(H)(h1, src2)
    msg2 = _edge_call(ea_t, hs, w1t_b, b1c_b, wbigt_b, bmt_b, H)
    msg3b = msg2.reshape(NS * NSUB_SC, SUB, 64)
    sums2, _ = _make_scatter(True)(msg3b, dst_sc, zeros64, zeros16, ones16)
    out = _node2_call(sums2, cnts, h1, root2, bias2.reshape(1, 64),
                      Wf, bf.reshape(1, F1))
    return out
